# baseline (speedup 1.0000x reference)
import functools

import jax
import jax.numpy as jnp
from jax import lax
from jax.experimental import pallas as pl
from jax.experimental.pallas import tpu as pltpu
from jax.experimental.pallas import tpu_sc as plsc

N = 16384
F = 26
V = 100000
D = 64
S = 768
K = F * D

_NC = 2
_NS = 16
_NW = _NC * _NS
_C = 1024
_G = 128
_TASKS = F * (N // _C)
_TPW = _TASKS // _NW


def _sc_gather_body(tables_hbm, idx_hbm, out_hbm, idx_v, rows_v, sem):
    wid = lax.axis_index("s") * _NC + lax.axis_index("c")

    def task(t_i, carry):
        t = wid * _TPW + t_i
        f = t // (N // _C)
        n0 = (t % (N // _C)) * _C
        pltpu.sync_copy(idx_hbm.at[pl.ds(f * N + n0, _C)], idx_v)
        base = f * V

        def addb(i, c):
            sl = pl.ds(i * 16, 16)
            idx_v[sl] = idx_v[sl] + base
            return c

        lax.fori_loop(0, _C // 16, addb, 0)
        copies = [
            pltpu.async_copy(
                tables_hbm.at[idx_v.at[pl.ds(j * _G, _G)]],
                rows_v.at[pl.ds(j * _G, _G), :],
                sem,
            )
            for j in range(_C // _G)
        ]
        for cp in copies:
            cp.wait()
        pltpu.sync_copy(rows_v, out_hbm.at[pl.ds(n0, _C), pl.ds(f * D, D)])
        return carry

    lax.fori_loop(0, _TPW, task, 0)


_sc_gather = functools.partial(
    pl.kernel,
    out_type=jax.ShapeDtypeStruct((N, K), jnp.float32),
    mesh=plsc.VectorSubcoreMesh(core_axis_name="c", subcore_axis_name="s"),
    compiler_params=pltpu.CompilerParams(use_tc_tiling_on_sc=False),
    scratch_types=[
        pltpu.VMEM((_C,), jnp.int32),
        pltpu.VMEM((_C, D), jnp.float32),
        pltpu.SemaphoreType.DMA,
    ],
)(_sc_gather_body)


_BN = 512


def _mm_body(g_ref, s_ref, w_ref, b_ref, o_ref):
    acc = lax.dot_general(
        g_ref[...], w_ref[...],
        (((1,), (1,)), ((), ())),
        preferred_element_type=jnp.float32,
    )
    o_ref[:, :S] = s_ref[...]
    o_ref[:, S:] = acc + b_ref[...]


def kernel(sentence_embeddings, categorical_data, tables, W, b):
    tables_flat = tables.reshape(F * V, D)
    gathered = _sc_gather(tables_flat, categorical_data.reshape(F * N))
    out = pl.pallas_call(
        _mm_body,
        grid=(N // _BN,),
        in_specs=[
            pl.BlockSpec((_BN, K), lambda i: (i, 0)),
            pl.BlockSpec((_BN, S), lambda i: (i, 0)),
            pl.BlockSpec((S, K), lambda i: (0, 0)),
            pl.BlockSpec((1, S), lambda i: (0, 0)),
        ],
        out_specs=pl.BlockSpec((_BN, 2 * S), lambda i: (i, 0)),
        out_shape=jax.ShapeDtypeStruct((N, 2 * S), jnp.float32),
    )(gathered, sentence_embeddings, W, b.reshape(1, S))
    return out


# speedup vs baseline: 1.6979x; 1.6979x over previous
import functools

import jax
import jax.numpy as jnp
from jax import lax
from jax.experimental import pallas as pl
from jax.experimental.pallas import tpu as pltpu
from jax.experimental.pallas import tpu_sc as plsc

N = 16384
F = 26
V = 100000
D = 64
S = 768
K = F * D

_NC = 2
_NS = 16
_NW = _NC * _NS
_C = 256
_CPW = N // _C // _NW
_NPAIR = F // 2


def _sc_concat_body(emb_hbm, out_hbm, buf0, buf1, comb):
    wid = lax.axis_index("s") * _NC + lax.axis_index("c")

    for cc in range(_CPW):
        n0 = pl.multiple_of((wid * _CPW + cc) * _C, _C)
        for p in range(_NPAIR):
            f0, f1 = 2 * p, 2 * p + 1
            pltpu.sync_copy(emb_hbm.at[f0, pl.ds(n0, _C), :], buf0)
            pltpu.sync_copy(emb_hbm.at[f1, pl.ds(n0, _C), :], buf1)

            def stitch(i, c):
                for j in range(D // 16):
                    sl = pl.ds(j * 16, 16)
                    comb[i, sl] = buf0[i, sl]
                    comb[i, pl.ds(D + j * 16, 16)] = buf1[i, sl]
                return c

            lax.fori_loop(0, _C, stitch, 0)
            pltpu.sync_copy(
                comb, out_hbm.at[pl.ds(n0, _C), pl.ds(p * 128, 128)])


_sc_concat = functools.partial(
    pl.kernel,
    out_type=jax.ShapeDtypeStruct((N, K), jnp.float32),
    mesh=plsc.VectorSubcoreMesh(core_axis_name="c", subcore_axis_name="s"),
    scratch_types=[
        pltpu.VMEM((_C, D), jnp.float32),
        pltpu.VMEM((_C, D), jnp.float32),
        pltpu.VMEM((_C, 2 * D), jnp.float32),
    ],
)(_sc_concat_body)


_BN = 512


def _mm_body(g_ref, s_ref, w_ref, b_ref, o_ref):
    acc = lax.dot_general(
        g_ref[...], w_ref[...],
        (((1,), (1,)), ((), ())),
        preferred_element_type=jnp.float32,
    )
    o_ref[:, :S] = s_ref[...]
    o_ref[:, S:] = acc + b_ref[...]


def kernel(sentence_embeddings, categorical_data, tables, W, b):
    emb = jax.vmap(lambda t, i: jnp.take(t, i, axis=0))(
        tables, categorical_data)
    gathered = _sc_concat(emb)
    out = pl.pallas_call(
        _mm_body,
        grid=(N // _BN,),
        in_specs=[
            pl.BlockSpec((_BN, K), lambda i: (i, 0)),
            pl.BlockSpec((_BN, S), lambda i: (i, 0)),
            pl.BlockSpec((S, K), lambda i: (0, 0)),
            pl.BlockSpec((1, S), lambda i: (0, 0)),
        ],
        out_specs=pl.BlockSpec((_BN, 2 * S), lambda i: (i, 0)),
        out_shape=jax.ShapeDtypeStruct((N, 2 * S), jnp.float32),
    )(gathered, sentence_embeddings, W, b.reshape(1, S))
    return out
